# baseline (speedup 1.0000x reference)
import functools

import jax
import jax.numpy as jnp
from jax import lax
from jax.experimental import pallas as pl
from jax.experimental.pallas import tpu as pltpu
from jax.experimental.pallas import tpu_sc as plsc

_NW = 32
_EXTENT = 1.2
_K = 15
_KN = 32


def _lrelu(t):
    return jnp.where(t > 0, t, 0.1 * t)


def _pick_chunk(b_per_w):
    for c in range(128, 7, -8):
        if b_per_w % c == 0:
            return c
    raise ValueError(f"no chunk for {b_per_w}")


def _sc_gather(table, idx):
    V, D = table.shape
    B = idx.shape[0]
    b_per_w = B // _NW
    chunk = _pick_chunk(b_per_w)
    n_chunks = b_per_w // chunk
    n_pairs = n_chunks // 2
    has_tail = n_chunks % 2 == 1
    mesh = plsc.VectorSubcoreMesh(core_axis_name="c", subcore_axis_name="s")

    @functools.partial(
        pl.kernel,
        out_type=jax.ShapeDtypeStruct((B, D), jnp.float32),
        mesh=mesh,
        scratch_types=[
            pltpu.VMEM((b_per_w,), jnp.int32),
            pltpu.VMEM((chunk, D), jnp.float32),
            pltpu.VMEM((chunk, D), jnp.float32),
            pltpu.SemaphoreType.DMA,
            pltpu.SemaphoreType.DMA,
            pltpu.SemaphoreType.DMA,
            pltpu.SemaphoreType.DMA,
        ],
        compiler_params=pltpu.CompilerParams(use_tc_tiling_on_sc=False),
    )
    def gather_kernel(table_hbm, idx_hbm, out_hbm, idx_v, buf0, buf1,
                      sg0, sg1, sw0, sw1):
        wid = lax.axis_index("s") * 2 + lax.axis_index("c")
        base = wid * b_per_w
        pltpu.sync_copy(idx_hbm.at[pl.ds(base, b_per_w)], idx_v)

        def gsrc(ci):
            return table_hbm.at[idx_v.at[pl.ds(ci * chunk, chunk)]]

        def dst(ci):
            return out_hbm.at[pl.ds(base + ci * chunk, chunk)]

        def body(g, carry):
            i0 = 2 * g
            i1 = i0 + 1

            @pl.when(g > 0)
            def _():
                pltpu.make_async_copy(buf0, dst(i0), sw0).wait()

            pltpu.async_copy(gsrc(i0), buf0, sg0)

            @pl.when(g > 0)
            def _():
                pltpu.make_async_copy(buf1, dst(i1), sw1).wait()

            pltpu.async_copy(gsrc(i1), buf1, sg1)
            pltpu.make_async_copy(gsrc(i0), buf0, sg0).wait()
            pltpu.async_copy(buf0, dst(i0), sw0)
            pltpu.make_async_copy(gsrc(i1), buf1, sg1).wait()
            pltpu.async_copy(buf1, dst(i1), sw1)
            return carry

        lax.fori_loop(0, n_pairs, body, 0)
        pltpu.make_async_copy(buf0, dst(0), sw0).wait()
        pltpu.make_async_copy(buf1, dst(0), sw1).wait()
        if has_tail:
            t = n_chunks - 1
            pltpu.async_copy(gsrc(t), buf0, sg0)
            pltpu.make_async_copy(gsrc(t), buf0, sg0).wait()
            pltpu.sync_copy(buf0, dst(t))

    return gather_kernel(table, idx)


def _infl(g, co, pts, kpt):
    d2 = None
    for c in range(3):
        nb = g[:, :, co + c:co + c + 1] - pts[:, c:c + 1][:, None, :]
        t = nb - kpt[:, c:c + 1, :]
        t = t * t
        d2 = t if d2 is None else d2 + t
    d = jnp.sqrt(d2 + 1e-9)
    return jnp.maximum(1.0 - d * (1.0 / _EXTENT), 0.0)


def _dot(a, b):
    return jnp.dot(a, b, preferred_element_type=jnp.float32)


def _kpconv(infl, feat, w_ref, d):
    pack = max(1, 128 // d)
    featrep = feat if pack == 1 else jnp.concatenate([feat] * pack, axis=2)
    b, kn = infl.shape[0], infl.shape[1]
    cols = []
    p = 0
    while p < _K:
        g = min(pack, _K - p)
        parts = [jnp.broadcast_to(infl[:, :, q:q + 1], (b, kn, d))
                 for q in range(p, p + g)]
        iq = jnp.concatenate(parts, axis=2) if g > 1 else parts[0]
        cols.append(jnp.sum(iq * featrep[:, :, :g * d], axis=1))
        p += g
    wxc = jnp.concatenate(cols, axis=1)
    return _dot(wxc, w_ref[...])


def _t1_body(g1, p0, kpt, wsf, u10, x64_o, x1_o, infl_o):
    infl = _infl(g1[...], 0, p0[...], kpt[...])
    x = _lrelu(_kpconv(infl, g1[...], wsf, 16))
    x64_o[...] = x
    x1_o[...] = _lrelu(_dot(x, u10[...]))
    b = infl.shape[0]
    infl_o[...] = jnp.concatenate([infl, jnp.zeros((b, _KN, 1), jnp.float32)], axis=2)


def _t2_body(g2, infl0, x64, p0, wk0, u20, ush0, u1s, tC_o):
    infl = infl0[:, :, 0:_K]
    x2 = _lrelu(_kpconv(infl, g2[...], wk0, 32))
    xr = _lrelu(_dot(x2, u20[...]) + _dot(x64[...], ush0[...]))
    y1 = _lrelu(_dot(xr, u1s[...]))
    b = xr.shape[0]
    tC_o[...] = jnp.concatenate(
        [xr, y1, p0[...], jnp.zeros((b, 13), jnp.float32)], axis=1)


def _t3_body(g3, p1, kpt, wks, u2s, ushs, u11, xl1_o, tD_o):
    xg = g3[:, :, 0:128]
    y1g = g3[:, :, 128:192]
    infl = _infl(g3[...], 192, p1[...], kpt[...])
    y2 = _lrelu(_kpconv(infl, y1g, wks, 64))
    mp = jnp.max(xg, axis=1)
    x = _lrelu(_dot(y2, u2s[...]) + _dot(mp, ushs[...]))
    xl1_o[...] = x
    z1 = _lrelu(_dot(x, u11[...]))
    b = x.shape[0]
    tD_o[...] = jnp.concatenate(
        [z1, p1[...], jnp.zeros((b, 13), jnp.float32)], axis=1)


def _t4_body(g4, xl1, p1, kpt, wk1, u21, xo_o):
    z1g = g4[:, :, 0:64]
    infl = _infl(g4[...], 64, p1[...], kpt[...])
    z2 = _lrelu(_kpconv(infl, z1g, wk1, 64))
    xo_o[...] = _lrelu(_dot(z2, u21[...]) + xl1[...])


def _t5_body(up, tC, udu, uds, wmlp, bmlp, wc, bc, ws, bs, s_o, c_o, f_o):
    skip = tC[:, 0:128]
    x = _lrelu(_dot(up[...], udu[...]) + _dot(skip, uds[...]))
    f = _lrelu(_dot(x, wmlp[...]) + bmlp[...])
    f_o[...] = f
    cl = _dot(f, wc[...]) + bc[...]
    c_o[...] = 1.0 / (1.0 + jnp.exp(-cl))
    s_o[...] = _dot(f, ws[...]) + bs[...]


def _full(shape):
    nd = len(shape)
    return pl.BlockSpec(shape, lambda i: (0,) * nd)


def _rows(b, *rest):
    nd = 1 + len(rest)
    return pl.BlockSpec((b, *rest), lambda i: (i,) + (0,) * (len(rest)))


def kernel(features, points0, points1, neighbors0, neighbors1, pools1,
           upsamples0, kernel_points, W_simple, U1_0, Wk_0, U2_0, Ush_0,
           U1_s, Wk_s, U2_s, Ush_s, U1_1, Wk_1, U2_1, Udec, Wmlp, bmlp,
           Wc, bc, Wv, bv, Ws, bs):
    f32, i32 = jnp.float32, jnp.int32
    N0, KN = neighbors0.shape
    N1 = points1.shape[0]
    K = kernel_points.shape[0]
    N1p = 2560
    b0, b1 = 200, 128
    kpt = kernel_points.T.reshape(1, 3, K)

    table0 = jnp.concatenate(
        [points0, features, jnp.zeros((N0, 8), f32)], axis=1)
    nb0 = neighbors0.reshape(-1).astype(i32)
    g1 = _sc_gather(table0, nb0).reshape(N0, KN, 16)

    wsf = jnp.zeros((K, 16, 64), f32).at[:, 3:8, :].set(W_simple).reshape(K * 16, 64)
    x64, x1, infl0 = pl.pallas_call(
        _t1_body,
        grid=(N0 // b0,),
        in_specs=[_rows(b0, KN, 16), _rows(b0, 3), _full((1, 3, K)),
                  _full((K * 16, 64)), _full((64, 32))],
        out_specs=[_rows(b0, 64), _rows(b0, 32), _rows(b0, KN, 16)],
        out_shape=[jax.ShapeDtypeStruct((N0, 64), f32),
                   jax.ShapeDtypeStruct((N0, 32), f32),
                   jax.ShapeDtypeStruct((N0, KN, 16), f32)],
    )(g1, points0, kpt, wsf, U1_0)

    g2 = _sc_gather(x1, nb0).reshape(N0, KN, 32)
    tC = pl.pallas_call(
        _t2_body,
        grid=(N0 // b0,),
        in_specs=[_rows(b0, KN, 32), _rows(b0, KN, 16), _rows(b0, 64),
                  _rows(b0, 3), _full((K * 32, 32)), _full((32, 128)),
                  _full((64, 128)), _full((128, 64))],
        out_specs=[_rows(b0, 208)],
        out_shape=[jax.ShapeDtypeStruct((N0, 208), f32)],
    )(g2, infl0, x64, points0, Wk_0.reshape(K * 32, 32), U2_0, Ush_0, U1_s)[0]

    pools_p = jnp.concatenate(
        [pools1.astype(i32), jnp.zeros((N1p - N1, KN), i32)], axis=0)
    g3 = _sc_gather(tC, pools_p.reshape(-1)).reshape(N1p, KN, 208)
    p1p = jnp.concatenate([points1, jnp.zeros((N1p - N1, 3), f32)], axis=0)
    xl1, tD = pl.pallas_call(
        _t3_body,
        grid=(N1p // b1,),
        in_specs=[_rows(b1, KN, 208), _rows(b1, 3), _full((1, 3, K)),
                  _full((K * 64, 64)), _full((64, 256)), _full((128, 256)),
                  _full((256, 64))],
        out_specs=[_rows(b1, 256), _rows(b1, 80)],
        out_shape=[jax.ShapeDtypeStruct((N1p, 256), f32),
                   jax.ShapeDtypeStruct((N1p, 80), f32)],
    )(g3, p1p, kpt, Wk_s.reshape(K * 64, 64), U2_s, Ush_s, U1_1)

    nb1_p = jnp.concatenate(
        [neighbors1.astype(i32), jnp.zeros((N1p - N1, KN), i32)], axis=0)
    g4 = _sc_gather(tD, nb1_p.reshape(-1)).reshape(N1p, KN, 80)
    xo1 = pl.pallas_call(
        _t4_body,
        grid=(N1p // b1,),
        in_specs=[_rows(b1, KN, 80), _rows(b1, 256), _rows(b1, 3),
                  _full((1, 3, K)), _full((K * 64, 64)), _full((64, 256))],
        out_specs=[_rows(b1, 256)],
        out_shape=[jax.ShapeDtypeStruct((N1p, 256), f32)],
    )(g4, xl1, p1p, kpt, Wk_1.reshape(K * 64, 64), U2_1)[0]

    B5 = 10240
    up_idx = jnp.concatenate(
        [upsamples0[:, 0].astype(i32), jnp.zeros((B5 - N0,), i32)], axis=0)
    g5 = _sc_gather(xo1, up_idx)
    s_out, c, f = pl.pallas_call(
        _t5_body,
        grid=(N0 // b0,),
        in_specs=[_rows(b0, 256), _rows(b0, 208), _full((256, 128)),
                  _full((128, 128)), _full((128, 128)), _full((1, 128)),
                  _full((128, 1)), _full((1, 1)), _full((128, 19)),
                  _full((1, 19))],
        out_specs=[_rows(b0, 19), _rows(b0, 1), _rows(b0, 128)],
        out_shape=[jax.ShapeDtypeStruct((N0, 19), f32),
                   jax.ShapeDtypeStruct((N0, 1), f32),
                   jax.ShapeDtypeStruct((N0, 128), f32)],
    )(g5, tC, Udec[0:256], Udec[256:384], Wmlp, bmlp.reshape(1, 128),
      Wc, bc.reshape(1, 1), Ws, bs.reshape(1, 19))
    return (s_out, c, f)


# speedup vs baseline: 1.1998x; 1.1998x over previous
import functools

import jax
import jax.numpy as jnp
from jax import lax
from jax.experimental import pallas as pl
from jax.experimental.pallas import tpu as pltpu
from jax.experimental.pallas import tpu_sc as plsc

_NW = 32
_EXTENT = 1.2
_K = 15
_KN = 32


def _lrelu(t):
    return jnp.where(t > 0, t, 0.1 * t)


def _pick_chunk(b_per_w):
    for c in range(128, 7, -8):
        if b_per_w % c == 0:
            return c
    raise ValueError(f"no chunk for {b_per_w}")


def _sc_gather(table, idx):
    V, D = table.shape
    B = idx.shape[0]
    b_per_w = B // _NW
    chunk = _pick_chunk(b_per_w)
    n_chunks = b_per_w // chunk
    n_pairs = n_chunks // 2
    has_tail = n_chunks % 2 == 1
    mesh = plsc.VectorSubcoreMesh(core_axis_name="c", subcore_axis_name="s")

    @functools.partial(
        pl.kernel,
        out_type=jax.ShapeDtypeStruct((B, D), jnp.float32),
        mesh=mesh,
        scratch_types=[
            pltpu.VMEM((b_per_w,), jnp.int32),
            pltpu.VMEM((chunk, D), jnp.float32),
            pltpu.VMEM((chunk, D), jnp.float32),
            pltpu.SemaphoreType.DMA,
            pltpu.SemaphoreType.DMA,
            pltpu.SemaphoreType.DMA,
            pltpu.SemaphoreType.DMA,
        ],
        compiler_params=pltpu.CompilerParams(use_tc_tiling_on_sc=False),
    )
    def gather_kernel(table_hbm, idx_hbm, out_hbm, idx_v, buf0, buf1,
                      sg0, sg1, sw0, sw1):
        wid = lax.axis_index("s") * 2 + lax.axis_index("c")
        base = wid * b_per_w
        pltpu.sync_copy(idx_hbm.at[pl.ds(base, b_per_w)], idx_v)

        def gsrc(ci):
            return table_hbm.at[idx_v.at[pl.ds(ci * chunk, chunk)]]

        def dst(ci):
            return out_hbm.at[pl.ds(base + ci * chunk, chunk)]

        def body(g, carry):
            i0 = 2 * g
            i1 = i0 + 1

            @pl.when(g > 0)
            def _():
                pltpu.make_async_copy(buf0, dst(i0), sw0).wait()

            pltpu.async_copy(gsrc(i0), buf0, sg0)

            @pl.when(g > 0)
            def _():
                pltpu.make_async_copy(buf1, dst(i1), sw1).wait()

            pltpu.async_copy(gsrc(i1), buf1, sg1)
            pltpu.make_async_copy(gsrc(i0), buf0, sg0).wait()
            pltpu.async_copy(buf0, dst(i0), sw0)
            pltpu.make_async_copy(gsrc(i1), buf1, sg1).wait()
            pltpu.async_copy(buf1, dst(i1), sw1)
            return carry

        lax.fori_loop(0, n_pairs, body, 0)
        pltpu.make_async_copy(buf0, dst(0), sw0).wait()
        pltpu.make_async_copy(buf1, dst(0), sw1).wait()
        if has_tail:
            t = n_chunks - 1
            pltpu.async_copy(gsrc(t), buf0, sg0)
            pltpu.make_async_copy(gsrc(t), buf0, sg0).wait()
            pltpu.sync_copy(buf0, dst(t))

    return gather_kernel(table, idx)


def _infl(g, co, pts, kpt):
    d2 = None
    for c in range(3):
        nb = g[:, :, co + c:co + c + 1] - pts[:, c:c + 1][:, None, :]
        t = nb - kpt[:, c:c + 1, :]
        t = t * t
        d2 = t if d2 is None else d2 + t
    d = jnp.sqrt(d2 + 1e-9)
    return jnp.maximum(1.0 - d * (1.0 / _EXTENT), 0.0)


def _dot(a, b):
    return jnp.dot(a, b, preferred_element_type=jnp.float32)


def _kpconv(infl, feat, w_ref, d):
    acc = None
    for p in range(_K):
        wp = jnp.sum(infl[:, :, p:p + 1] * feat, axis=1)
        t = _dot(wp, w_ref[p * d:(p + 1) * d, :])
        acc = t if acc is None else acc + t
    return acc


def _t1_body(g1, p0, kpt, wsf, u10, x64_o, x1_o, infl_o):
    infl = _infl(g1[...], 0, p0[...], kpt[...])
    x = _lrelu(_kpconv(infl, g1[...], wsf, 16))
    x64_o[...] = x
    x1_o[...] = _lrelu(_dot(x, u10[...]))
    b = infl.shape[0]
    infl_o[...] = jnp.concatenate([infl, jnp.zeros((b, _KN, 1), jnp.float32)], axis=2)


def _t2_body(g2, infl0, x64, p0, wk0, u20, ush0, u1s, tC_o):
    infl = infl0[:, :, 0:_K]
    x2 = _lrelu(_kpconv(infl, g2[...], wk0, 32))
    xr = _lrelu(_dot(x2, u20[...]) + _dot(x64[...], ush0[...]))
    y1 = _lrelu(_dot(xr, u1s[...]))
    b = xr.shape[0]
    tC_o[...] = jnp.concatenate(
        [xr, y1, p0[...], jnp.zeros((b, 13), jnp.float32)], axis=1)


def _t3_body(g3, p1, kpt, wks, u2s, ushs, u11, xl1_o, tD_o):
    xg = g3[:, :, 0:128]
    y1g = g3[:, :, 128:192]
    infl = _infl(g3[...], 192, p1[...], kpt[...])
    y2 = _lrelu(_kpconv(infl, y1g, wks, 64))
    mp = jnp.max(xg, axis=1)
    x = _lrelu(_dot(y2, u2s[...]) + _dot(mp, ushs[...]))
    xl1_o[...] = x
    z1 = _lrelu(_dot(x, u11[...]))
    b = x.shape[0]
    tD_o[...] = jnp.concatenate(
        [z1, p1[...], jnp.zeros((b, 13), jnp.float32)], axis=1)


def _t4_body(g4, xl1, p1, kpt, wk1, u21, xo_o):
    z1g = g4[:, :, 0:64]
    infl = _infl(g4[...], 64, p1[...], kpt[...])
    z2 = _lrelu(_kpconv(infl, z1g, wk1, 64))
    xo_o[...] = _lrelu(_dot(z2, u21[...]) + xl1[...])


def _t5_body(up, tC, udu, uds, wmlp, bmlp, wc, bc, ws, bs, s_o, c_o, f_o):
    skip = tC[:, 0:128]
    x = _lrelu(_dot(up[...], udu[...]) + _dot(skip, uds[...]))
    f = _lrelu(_dot(x, wmlp[...]) + bmlp[...])
    f_o[...] = f
    cl = _dot(f, wc[...]) + bc[...]
    c_o[...] = 1.0 / (1.0 + jnp.exp(-cl))
    s_o[...] = _dot(f, ws[...]) + bs[...]


def _full(shape):
    nd = len(shape)
    return pl.BlockSpec(shape, lambda i: (0,) * nd)


def _rows(b, *rest):
    nd = 1 + len(rest)
    return pl.BlockSpec((b, *rest), lambda i: (i,) + (0,) * (len(rest)))


def kernel(features, points0, points1, neighbors0, neighbors1, pools1,
           upsamples0, kernel_points, W_simple, U1_0, Wk_0, U2_0, Ush_0,
           U1_s, Wk_s, U2_s, Ush_s, U1_1, Wk_1, U2_1, Udec, Wmlp, bmlp,
           Wc, bc, Wv, bv, Ws, bs):
    f32, i32 = jnp.float32, jnp.int32
    N0, KN = neighbors0.shape
    N1 = points1.shape[0]
    K = kernel_points.shape[0]
    N1p = 2560
    b0, b1 = 200, 128
    kpt = kernel_points.T.reshape(1, 3, K)

    table0 = jnp.concatenate(
        [points0, features, jnp.zeros((N0, 8), f32)], axis=1)
    nb0 = neighbors0.reshape(-1).astype(i32)
    g1 = _sc_gather(table0, nb0).reshape(N0, KN, 16)

    wsf = jnp.zeros((K, 16, 64), f32).at[:, 3:8, :].set(W_simple).reshape(K * 16, 64)
    x64, x1, infl0 = pl.pallas_call(
        _t1_body,
        grid=(N0 // b0,),
        in_specs=[_rows(b0, KN, 16), _rows(b0, 3), _full((1, 3, K)),
                  _full((K * 16, 64)), _full((64, 32))],
        out_specs=[_rows(b0, 64), _rows(b0, 32), _rows(b0, KN, 16)],
        out_shape=[jax.ShapeDtypeStruct((N0, 64), f32),
                   jax.ShapeDtypeStruct((N0, 32), f32),
                   jax.ShapeDtypeStruct((N0, KN, 16), f32)],
    )(g1, points0, kpt, wsf, U1_0)

    g2 = _sc_gather(x1, nb0).reshape(N0, KN, 32)
    tC = pl.pallas_call(
        _t2_body,
        grid=(N0 // b0,),
        in_specs=[_rows(b0, KN, 32), _rows(b0, KN, 16), _rows(b0, 64),
                  _rows(b0, 3), _full((K * 32, 32)), _full((32, 128)),
                  _full((64, 128)), _full((128, 64))],
        out_specs=[_rows(b0, 208)],
        out_shape=[jax.ShapeDtypeStruct((N0, 208), f32)],
    )(g2, infl0, x64, points0, Wk_0.reshape(K * 32, 32), U2_0, Ush_0, U1_s)[0]

    pools_p = jnp.concatenate(
        [pools1.astype(i32), jnp.zeros((N1p - N1, KN), i32)], axis=0)
    g3 = _sc_gather(tC, pools_p.reshape(-1)).reshape(N1p, KN, 208)
    p1p = jnp.concatenate([points1, jnp.zeros((N1p - N1, 3), f32)], axis=0)
    xl1, tD = pl.pallas_call(
        _t3_body,
        grid=(N1p // b1,),
        in_specs=[_rows(b1, KN, 208), _rows(b1, 3), _full((1, 3, K)),
                  _full((K * 64, 64)), _full((64, 256)), _full((128, 256)),
                  _full((256, 64))],
        out_specs=[_rows(b1, 256), _rows(b1, 80)],
        out_shape=[jax.ShapeDtypeStruct((N1p, 256), f32),
                   jax.ShapeDtypeStruct((N1p, 80), f32)],
    )(g3, p1p, kpt, Wk_s.reshape(K * 64, 64), U2_s, Ush_s, U1_1)

    nb1_p = jnp.concatenate(
        [neighbors1.astype(i32), jnp.zeros((N1p - N1, KN), i32)], axis=0)
    g4 = _sc_gather(tD, nb1_p.reshape(-1)).reshape(N1p, KN, 80)
    xo1 = pl.pallas_call(
        _t4_body,
        grid=(N1p // b1,),
        in_specs=[_rows(b1, KN, 80), _rows(b1, 256), _rows(b1, 3),
                  _full((1, 3, K)), _full((K * 64, 64)), _full((64, 256))],
        out_specs=[_rows(b1, 256)],
        out_shape=[jax.ShapeDtypeStruct((N1p, 256), f32)],
    )(g4, xl1, p1p, kpt, Wk_1.reshape(K * 64, 64), U2_1)[0]

    B5 = 10240
    up_idx = jnp.concatenate(
        [upsamples0[:, 0].astype(i32), jnp.zeros((B5 - N0,), i32)], axis=0)
    g5 = _sc_gather(xo1, up_idx)
    s_out, c, f = pl.pallas_call(
        _t5_body,
        grid=(N0 // b0,),
        in_specs=[_rows(b0, 256), _rows(b0, 208), _full((256, 128)),
                  _full((128, 128)), _full((128, 128)), _full((1, 128)),
                  _full((128, 1)), _full((1, 1)), _full((128, 19)),
                  _full((1, 19))],
        out_specs=[_rows(b0, 19), _rows(b0, 1), _rows(b0, 128)],
        out_shape=[jax.ShapeDtypeStruct((N0, 19), f32),
                   jax.ShapeDtypeStruct((N0, 1), f32),
                   jax.ShapeDtypeStruct((N0, 128), f32)],
    )(g5, tC, Udec[0:256], Udec[256:384], Wmlp, bmlp.reshape(1, 128),
      Wc, bc.reshape(1, 1), Ws, bs.reshape(1, 19))
    return (s_out, c, f)
